# baseline (speedup 1.0000x reference)
import functools

import jax
import jax.numpy as jnp
from jax import lax
from jax.experimental import pallas as pl
from jax.experimental.pallas import tpu as pltpu
from jax.experimental.pallas import tpu_sc as plsc

_NUM_CORES = 2
_NUM_SUBCORES = 16
_NUM_WORKERS = _NUM_CORES * _NUM_SUBCORES
_CHUNK = 128


def kernel(x, table):
    B, S = x.shape
    V, D = table.shape
    N = B * S
    assert N % (_NUM_WORKERS * _CHUNK) == 0
    per_w = N // _NUM_WORKERS
    n_chunks = per_w // _CHUNK

    idx = x.reshape(N)
    mesh = plsc.VectorSubcoreMesh(core_axis_name="c", subcore_axis_name="s")

    @functools.partial(
        pl.kernel,
        out_type=jax.ShapeDtypeStruct((N, D), jnp.float32),
        mesh=mesh,
        scratch_types=[
            pltpu.VMEM((_CHUNK,), jnp.int32),
            pltpu.VMEM((_CHUNK, D), jnp.float32),
            pltpu.SemaphoreType.DMA,
        ],
        compiler_params=pltpu.CompilerParams(use_tc_tiling_on_sc=False),
    )
    def emb(idx_hbm, table_hbm, out_hbm, idx_v, rows_v, sem):
        wid = lax.axis_index("s") * _NUM_CORES + lax.axis_index("c")
        base = wid * per_w

        @pl.loop(0, n_chunks)
        def _step(i):
            off = base + i * _CHUNK
            pltpu.sync_copy(idx_hbm.at[pl.ds(off, _CHUNK)], idx_v)
            pltpu.async_copy(table_hbm.at[idx_v], rows_v, sem).wait()
            pltpu.sync_copy(rows_v, out_hbm.at[pl.ds(off, _CHUNK)])

    out = emb(idx, table)
    return out.reshape(B, S, D)


# speedup vs baseline: 1.1936x; 1.1936x over previous
import functools

import jax
import jax.numpy as jnp
from jax import lax
from jax.experimental import pallas as pl
from jax.experimental.pallas import tpu as pltpu
from jax.experimental.pallas import tpu_sc as plsc

_NUM_CORES = 2
_NUM_SUBCORES = 16
_NUM_WORKERS = _NUM_CORES * _NUM_SUBCORES
_CHUNK = 128
_K = 4
_NBUF = 2


def kernel(x, table):
    B, S = x.shape
    V, D = table.shape
    N = B * S
    n_chunks = N // _CHUNK
    per_w_chunks = n_chunks // _NUM_WORKERS
    n_groups = per_w_chunks // _K
    assert n_chunks % _NUM_WORKERS == 0 and per_w_chunks % _K == 0

    idx = x.reshape(n_chunks, _CHUNK)
    mesh = plsc.VectorSubcoreMesh(core_axis_name="c", subcore_axis_name="s")

    @functools.partial(
        pl.kernel,
        out_type=jax.ShapeDtypeStruct((n_chunks, _CHUNK, D), jnp.float32),
        mesh=mesh,
        scratch_types=[
            pltpu.VMEM((per_w_chunks, _CHUNK), jnp.int32),
            pltpu.VMEM((_NBUF, _K, _CHUNK, D), jnp.float32),
            pltpu.SemaphoreType.DMA((_NBUF,)),
            pltpu.SemaphoreType.DMA((_NBUF,)),
        ],
        compiler_params=pltpu.CompilerParams(use_tc_tiling_on_sc=False),
    )
    def emb(idx_hbm, table_hbm, out_hbm, idx_v, rows_v, gsem, osem):
        wid = lax.axis_index("s") * _NUM_CORES + lax.axis_index("c")
        cbase = wid * per_w_chunks
        pltpu.sync_copy(idx_hbm.at[pl.ds(cbase, per_w_chunks)], idx_v)

        def fire_gathers(t, sl):
            for b in range(_K):
                pltpu.async_copy(
                    table_hbm.at[idx_v.at[t * _K + b]], rows_v.at[sl, b],
                    gsem.at[sl])

        def drain_gathers(sl):
            for b in range(_K):
                pltpu.make_async_copy(
                    table_hbm.at[pl.ds(0, _CHUNK)], rows_v.at[sl, b],
                    gsem.at[sl]).wait()

        def drain_out(sl):
            pltpu.make_async_copy(
                rows_v.at[sl], out_hbm.at[pl.ds(cbase, _K)],
                osem.at[sl]).wait()

        fire_gathers(0, 0)

        @pl.loop(0, n_groups, step=_NBUF)
        def _grp(t0):
            for sl in range(_NBUF):
                t = t0 + sl
                nsl = (sl + 1) % _NBUF

                @pl.when(t + 1 < n_groups)
                def _fire_next():
                    @pl.when(t >= 1)
                    def _drain_prev_out():
                        drain_out(nsl)
                    fire_gathers(t + 1, nsl)

                drain_gathers(sl)
                pltpu.async_copy(
                    rows_v.at[sl], out_hbm.at[pl.ds(cbase + t * _K, _K)],
                    osem.at[sl])

        for sl in range(_NBUF):
            drain_out(sl)

    out = emb(idx, table)
    return out.reshape(B, S, D)
